# baseline (speedup 1.0000x reference)
import functools

import jax
import jax.numpy as jnp
from jax import lax
from jax.experimental import pallas as pl
from jax.experimental.pallas import tpu as pltpu
from jax.experimental.pallas import tpu_sc as plsc

VOCAB = 100000
EMB = 300
NOUT = 2
BATCH = 4096
HIST = 50
LANES = 16
NC, NS = 2, 16
NW = NC * NS
NSHARD = NW // NOUT
B_PER_W = BATCH // NSHARD
NGRP = B_PER_W // LANES
K_BLK = 64
K_STEPS = -(-EMB // K_BLK)
K_PAD = K_BLK * K_STEPS
VH = 50048


def _matmul_body(wt_ref, w_ref, o0_ref, o1_ref):
    part = lax.dot_general(wt_ref[...], w_ref[...],
                           (((0,), (0,)), ((), ())),
                           preferred_element_type=jnp.float32)

    @pl.when(pl.program_id(0) == 0)
    def _():
        o0_ref[...] = part[0]
        o1_ref[...] = part[1]

    @pl.when(pl.program_id(0) > 0)
    def _():
        o0_ref[...] = o0_ref[...] + part[0]
        o1_ref[...] = o1_ref[...] + part[1]


def _project_half(wT, wtp, h):
    return pl.pallas_call(
        _matmul_body,
        grid=(K_STEPS,),
        in_specs=[
            pl.BlockSpec((K_BLK, 8), lambda i: (i, 0)),
            pl.BlockSpec((K_BLK, VH), lambda i, _h=h: (i, _h)),
        ],
        out_specs=[pl.BlockSpec((VH,), lambda i: (0,)),
                   pl.BlockSpec((VH,), lambda i: (0,))],
        out_shape=[jax.ShapeDtypeStruct((VH,), jnp.float32),
                   jax.ShapeDtypeStruct((VH,), jnp.float32)],
        compiler_params=pltpu.CompilerParams(vmem_limit_bytes=56 * 2**20),
    )(wtp, wT)


def _make_pool_body(base, with_partial):
    def body(p0_hbm, p1_hbm, xt_hbm, *rest):
        if with_partial:
            (pin_hbm, out_hbm, tbl_v, xt_v, out_v, pin_v, scale_v,
             tbl_sem, xt_sem) = rest
        else:
            (out_hbm, tbl_v, xt_v, out_v, scale_v, tbl_sem, xt_sem) = rest
        wid = lax.axis_index("s") * NC + lax.axis_index("c")
        col = wid % NOUT
        r0 = (wid // NOUT) * B_PER_W

        xt_copy = pltpu.async_copy(xt_hbm.at[:, pl.ds(r0, B_PER_W)], xt_v,
                                   xt_sem)

        @pl.when(col == 0)
        def _():
            pltpu.async_copy(p0_hbm, tbl_v, tbl_sem)

        @pl.when(col == 1)
        def _():
            pltpu.async_copy(p1_hbm, tbl_v, tbl_sem)
        if with_partial:
            pltpu.sync_copy(pin_hbm.at[col, pl.ds(r0, B_PER_W)], pin_v)
        scale_v[...] = jnp.full((LANES,), 1.0 / HIST, jnp.float32)
        xt_copy.wait()
        pltpu.make_async_copy(p0_hbm, tbl_v, tbl_sem).wait()

        zero = jnp.zeros((LANES,), jnp.float32)
        base_v = jnp.full((LANES,), base, jnp.int32)
        last_v = jnp.full((LANES,), VH - 1, jnp.int32)

        @pl.loop(0, NGRP)
        def _grp(g):
            acc = zero
            for l in range(HIST):
                idx = xt_v[l, pl.ds(g * LANES, LANES)] - base_v
                m = idx <= last_v if base == 0 else idx >= 0
                idx_c = (jnp.minimum(idx, last_v) if base == 0
                         else jnp.maximum(idx, 0))
                val = plsc.load_gather(tbl_v, [idx_c])
                acc = acc + jnp.where(m, val, zero)
            res = acc * scale_v[...]
            if with_partial:
                res = res + pin_v[pl.ds(g * LANES, LANES)]
            out_v[pl.ds(g * LANES, LANES)] = res

        pltpu.sync_copy(out_v, out_hbm.at[col, pl.ds(r0, B_PER_W)])

    return body


@functools.cache
def _pool(base, with_partial):
    scratch = [
        pltpu.VMEM((VH,), jnp.float32),
        pltpu.VMEM((HIST, B_PER_W), jnp.int32),
        pltpu.VMEM((B_PER_W,), jnp.float32),
    ]
    if with_partial:
        scratch.append(pltpu.VMEM((B_PER_W,), jnp.float32))
    scratch += [
        pltpu.VMEM((LANES,), jnp.float32),
        pltpu.SemaphoreType.DMA,
        pltpu.SemaphoreType.DMA,
    ]
    return pl.kernel(
        _make_pool_body(base, with_partial),
        out_type=jax.ShapeDtypeStruct((NOUT, BATCH), jnp.float32),
        mesh=plsc.VectorSubcoreMesh(core_axis_name="c", subcore_axis_name="s",
                                    num_cores=NC, num_subcores=NS),
        compiler_params=pltpu.CompilerParams(use_tc_tiling_on_sc=False,
                                             needs_layout_passes=False),
        scratch_types=scratch,
    )


def kernel(x, weight, W_out):
    wtp = jnp.zeros((K_PAD, 8), jnp.float32).at[:EMB, :NOUT].set(W_out.T)
    wT = weight.T
    xt = x.astype(jnp.int32).T
    p0a, p1a = _project_half(wT, wtp, 0)
    p0b, p1b = _project_half(wT, wtp, 1)
    partial = _pool(0, False)(p0a, p1a, xt)
    pooled = _pool(VH, True)(p0b, p1b, xt, partial)
    return pooled.T
